# Initial kernel scaffold; baseline (speedup 1.0000x reference)
import jax
import jax.numpy as jnp
from jax.experimental import pallas as pl


def kernel(features, node_type, adj, batch, type_embeds, W_self_0, W_neigh_0, W_self_1, W_neigh_1):
    raise NotImplementedError("write your pallas kernel here")


# speedup vs baseline: 2.8313x; 2.8313x over previous
import dataclasses
import functools

import jax
import jax.numpy as jnp
from jax import lax
from jax.experimental import pallas as pl
from jax.experimental.pallas import tpu as pltpu
from jax.experimental.pallas import tpu_sc as plsc

N = 100000
D = 128
B = 512
MAX_DEG = 25
S1, S2 = 25, 10
HID = 64
TD = 16

NW = 32
BPW = B // NW
S1W = BPW * S1
CH = 40
NCH = S1W // CH
LANES = 16


def _expand_ids(src_ref, dst_ref, count, fanout, src_off):
    lane = lax.iota(jnp.int32, LANES)
    for k in range((count * fanout) // LANES):
        p = k * LANES + lane
        e = p // fanout
        q = p - e * fanout
        e_lo, e_hi = (k * LANES) // fanout, (k * LANES + LANES - 1) // fanout
        if e_lo == e_hi:
            chunk = src_ref[pl.ds((e_lo // LANES) * LANES, LANES)]
            ids = jnp.sum(jnp.where(lane == e_lo % LANES, chunk, 0))
        else:
            ids = plsc.load_gather(src_ref, [src_off + e])
        dst_ref[pl.ds(k * LANES, LANES)] = ids * MAX_DEG + q


def _sc_body(feat_hbm, adjf_hbm, batch_hbm, nt_hbm,
             f0_hbm, f1_hbm, f2s_hbm, nt0_hbm, nt1_hbm, nt2_hbm,
             idx0_v, fidx_v, idx1_v, idx2_v,
             rows2_v, rows1_v, acc_v, nt1_v, nt2_v, f0_v, nt0_v):
    w = lax.axis_index("s") * 2 + lax.axis_index("c")
    base0 = w * BPW
    base1 = w * S1W
    base2 = w * S1W * S2

    pltpu.sync_copy(batch_hbm.at[pl.ds(base0, BPW)], idx0_v)
    pltpu.sync_copy(feat_hbm.at[idx0_v], f0_v)
    pltpu.sync_copy(nt_hbm.at[idx0_v], nt0_v)
    pltpu.sync_copy(f0_v, f0_hbm.at[pl.ds(base0, BPW)])
    pltpu.sync_copy(nt0_v, nt0_hbm.at[pl.ds(base0, BPW)])

    _expand_ids(idx0_v, fidx_v, BPW, S1, 0)
    pltpu.sync_copy(adjf_hbm.at[fidx_v], idx1_v)
    pltpu.sync_copy(nt_hbm.at[idx1_v], nt1_v)
    pltpu.sync_copy(nt1_v, nt1_hbm.at[pl.ds(base1, S1W)])

    @pl.loop(0, NCH)
    def _chunk(cidx):
        off = cidx * CH
        idx1_c = idx1_v.at[pl.ds(off, CH)]
        _expand_ids(idx1_v, fidx_v, CH, S2, off)
        pltpu.sync_copy(adjf_hbm.at[fidx_v], idx2_v)

        pltpu.sync_copy(nt_hbm.at[idx2_v], nt2_v)
        pltpu.sync_copy(nt2_v, nt2_hbm.at[pl.ds(base2 + cidx * CH * S2, CH * S2)])

        pltpu.sync_copy(feat_hbm.at[idx1_c], rows1_v)
        pltpu.sync_copy(rows1_v, f1_hbm.at[pl.ds(base1 + off, CH)])

        pltpu.sync_copy(feat_hbm.at[idx2_v], rows2_v)

        @pl.loop(0, CH)
        def _acc(e):
            r = e * S2
            for dlo in range(D // LANES):
                sl = pl.ds(dlo * LANES, LANES)
                v = rows2_v[r, sl]
                for q in range(1, S2):
                    v = v + rows2_v[r + q, sl]
                acc_v[e, sl] = v

        pltpu.sync_copy(acc_v, f2s_hbm.at[pl.ds(base1 + off, CH)])


def _sc_gather(features, adj, batch, node_type):
    mesh = plsc.VectorSubcoreMesh(core_axis_name="c", subcore_axis_name="s")
    f32, i32 = jnp.float32, jnp.int32
    kern = pl.kernel(
        _sc_body,
        compiler_params=dataclasses.replace(
            pltpu.CompilerParams(), needs_layout_passes=False),
        out_type=[
            jax.ShapeDtypeStruct((B, D), f32),
            jax.ShapeDtypeStruct((B * S1, D), f32),
            jax.ShapeDtypeStruct((B * S1, D), f32),
            jax.ShapeDtypeStruct((B,), i32),
            jax.ShapeDtypeStruct((B * S1,), i32),
            jax.ShapeDtypeStruct((B * S1 * S2,), i32),
        ],
        mesh=mesh,
        scratch_types=[
            pltpu.VMEM((BPW,), i32),
            pltpu.VMEM((S1W,), i32),
            pltpu.VMEM((S1W,), i32),
            pltpu.VMEM((CH * S2,), i32),
            pltpu.VMEM((CH * S2, D), f32),
            pltpu.VMEM((CH, D), f32),
            pltpu.VMEM((CH, D), f32),
            pltpu.VMEM((S1W,), i32),
            pltpu.VMEM((CH * S2,), i32),
            pltpu.VMEM((BPW, D), f32),
            pltpu.VMEM((BPW,), i32),
        ],
    )
    return kern(features, adj, batch, node_type)


R1BLK = 1600
G1BLK = R1BLK // S1


def _tc1_body(f1_r, f2s_r, nt1_r, nt2_r, te_r, ws0_r, wn0_r,
              h1m_r, f1m_r, te1m_r):
    f32 = jnp.float32
    te = te_r[...]
    ws0 = ws0_r[...]
    wn0 = wn0_r[...]
    wsf, wst = ws0[:D], ws0[D:D + TD] + ws0[D + TD:]
    wnf, wnt = wn0[:D], wn0[D:D + TD] + wn0[D + TD:]

    nt1 = nt1_r[...]
    te1 = jnp.zeros((R1BLK, TD), f32)
    for t in range(4):
        te1 = te1 + jnp.where(nt1 == t, 1.0, 0.0) * te[t][None, :]

    nt2 = nt2_r[...]
    te2s = jnp.zeros((R1BLK, TD), f32)
    for t in range(4):
        cnt = jnp.sum(jnp.where(nt2 == t, 1.0, 0.0), axis=1, keepdims=True)
        te2s = te2s + cnt * te[t][None, :]

    f1 = f1_r[...]
    p_s = (jnp.dot(f1, wsf, preferred_element_type=f32)
           + jnp.dot(te1, wst, preferred_element_type=f32))
    p_n = (jnp.dot(f2s_r[...] * (1.0 / S2), wnf, preferred_element_type=f32)
           + jnp.dot(te2s * (1.0 / S2), wnt, preferred_element_type=f32))
    h1 = jnp.maximum(jnp.concatenate([p_s, p_n], axis=1), 0.0)

    h1m_r[...] = jnp.mean(h1.reshape(G1BLK, S1, 2 * HID), axis=1)
    f1m_r[...] = jnp.mean(f1.reshape(G1BLK, S1, D), axis=1)
    te1m_r[...] = jnp.mean(te1.reshape(G1BLK, S1, TD), axis=1)


def _tc_stage1(f1, f2s, nt1, nt2, te, ws0, wn0):
    f32 = jnp.float32
    n_steps = (B * S1) // R1BLK
    full = lambda shape: pl.BlockSpec(shape, lambda i: (0, 0))
    return pl.pallas_call(
        _tc1_body,
        grid=(n_steps,),
        in_specs=[
            pl.BlockSpec((R1BLK, D), lambda i: (i, 0)),
            pl.BlockSpec((R1BLK, D), lambda i: (i, 0)),
            pl.BlockSpec((R1BLK, 1), lambda i: (i, 0)),
            pl.BlockSpec((R1BLK, S2), lambda i: (i, 0)),
            full((4, TD)),
            full((D + 2 * TD, HID)),
            full((D + 2 * TD, HID)),
        ],
        out_specs=[
            pl.BlockSpec((G1BLK, 2 * HID), lambda i: (i, 0)),
            pl.BlockSpec((G1BLK, D), lambda i: (i, 0)),
            pl.BlockSpec((G1BLK, TD), lambda i: (i, 0)),
        ],
        out_shape=[
            jax.ShapeDtypeStruct((B, 2 * HID), f32),
            jax.ShapeDtypeStruct((B, D), f32),
            jax.ShapeDtypeStruct((B, TD), f32),
        ],
    )(f1, f2s, nt1, nt2, te, ws0, wn0)


def _tc2_body(f0_r, nt0_r, h1m_r, f1m_r, te1m_r, te_r,
              ws0_r, wn0_r, ws1_r, wn1_r, out_r):
    f32 = jnp.float32
    te = te_r[...]
    ws0 = ws0_r[...]
    wn0 = wn0_r[...]
    wsf, wst = ws0[:D], ws0[D:D + TD] + ws0[D + TD:]
    wnf, wnt = wn0[:D], wn0[D:D + TD] + wn0[D + TD:]

    nt0 = nt0_r[...]
    te0 = jnp.zeros((B, TD), f32)
    for t in range(4):
        te0 = te0 + jnp.where(nt0 == t, 1.0, 0.0) * te[t][None, :]

    p_s = (jnp.dot(f0_r[...], wsf, preferred_element_type=f32)
           + jnp.dot(te0, wst, preferred_element_type=f32))
    p_n = (jnp.dot(f1m_r[...], wnf, preferred_element_type=f32)
           + jnp.dot(te1m_r[...], wnt, preferred_element_type=f32))
    h0 = jnp.maximum(jnp.concatenate([p_s, p_n], axis=1), 0.0)

    out = jnp.concatenate(
        [jnp.dot(h0, ws1_r[...], preferred_element_type=f32),
         jnp.dot(h1m_r[...], wn1_r[...], preferred_element_type=f32)], axis=1)
    norm = jnp.sqrt(jnp.sum(out * out, axis=1, keepdims=True))
    out_r[...] = out / (norm + 1e-12)


def _tc_stage2(f0, nt0, h1m, f1m, te1m, te, ws0, wn0, ws1, wn1):
    return pl.pallas_call(
        _tc2_body,
        out_shape=jax.ShapeDtypeStruct((B, D), jnp.float32),
    )(f0, nt0, h1m, f1m, te1m, te, ws0, wn0, ws1, wn1)


def kernel(features, node_type, adj, batch, type_embeds,
           W_self_0, W_neigh_0, W_self_1, W_neigh_1):
    nt = node_type.astype(jnp.int32)
    adj_i = adj.astype(jnp.int32)
    batch_i = batch.astype(jnp.int32)

    f0, f1, f2s, nt0, nt1, nt2 = _sc_gather(
        features, adj_i.reshape(-1), batch_i, nt)

    h1m, f1m, te1m = _tc_stage1(
        f1, f2s, nt1.reshape(B * S1, 1), nt2.reshape(B * S1, S2),
        type_embeds, W_self_0, W_neigh_0)

    return _tc_stage2(f0, nt0.reshape(B, 1), h1m, f1m, te1m,
                      type_embeds, W_self_0, W_neigh_0, W_self_1, W_neigh_1)


# speedup vs baseline: 3.0033x; 1.0607x over previous
import dataclasses
import functools

import jax
import jax.numpy as jnp
from jax import lax
from jax.experimental import pallas as pl
from jax.experimental.pallas import tpu as pltpu
from jax.experimental.pallas import tpu_sc as plsc

N = 100000
D = 128
B = 512
MAX_DEG = 25
S1, S2 = 25, 10
HID = 64
TD = 16

ADJP = 128
NW = 32
BPW = B // NW
S1W = BPW * S1
CH = 40
NCH = S1W // CH
LANES = 16


def _flatten_rows(src_ref, dst_ref, count, fanout):
    lane = lax.iota(jnp.int32, LANES)
    for k in range((count * fanout) // LANES):
        p = k * LANES + lane
        e = p // fanout
        q = p - e * fanout
        dst_ref[pl.ds(k * LANES, LANES)] = plsc.load_gather(src_ref, [e, q])


def _sc_body(feat_hbm, adj_hbm, batch_hbm, nt_hbm,
             f0_hbm, f1_hbm, f2s_hbm, nt0_hbm, nt1_hbm, nt2_hbm,
             idx0_v, adjr0_v, idx1_v, adjr1_v, idx2_v,
             rows2_v, rows1_v, acc_v, nt1_v, nt2_v, f0_v, nt0_v):
    w = lax.axis_index("s") * 2 + lax.axis_index("c")
    base0 = w * BPW
    base1 = w * S1W
    base2 = w * S1W * S2

    pltpu.sync_copy(batch_hbm.at[pl.ds(base0, BPW)], idx0_v)
    pltpu.sync_copy(feat_hbm.at[idx0_v], f0_v)
    pltpu.sync_copy(nt_hbm.at[idx0_v], nt0_v)
    pltpu.sync_copy(f0_v, f0_hbm.at[pl.ds(base0, BPW)])
    pltpu.sync_copy(nt0_v, nt0_hbm.at[pl.ds(base0, BPW)])

    pltpu.sync_copy(adj_hbm.at[idx0_v, :], adjr0_v)
    _flatten_rows(adjr0_v, idx1_v, BPW, S1)
    pltpu.sync_copy(nt_hbm.at[idx1_v], nt1_v)
    pltpu.sync_copy(nt1_v, nt1_hbm.at[pl.ds(base1, S1W)])

    @pl.loop(0, NCH)
    def _chunk(cidx):
        off = cidx * CH
        idx1_c = idx1_v.at[pl.ds(off, CH)]
        pltpu.sync_copy(adj_hbm.at[idx1_c, :], adjr1_v)
        _flatten_rows(adjr1_v, idx2_v, CH, S2)

        pltpu.sync_copy(nt_hbm.at[idx2_v], nt2_v)
        pltpu.sync_copy(nt2_v, nt2_hbm.at[pl.ds(base2 + cidx * CH * S2, CH * S2)])

        pltpu.sync_copy(feat_hbm.at[idx1_c], rows1_v)
        pltpu.sync_copy(rows1_v, f1_hbm.at[pl.ds(base1 + off, CH)])

        pltpu.sync_copy(feat_hbm.at[idx2_v], rows2_v)

        @pl.loop(0, CH)
        def _acc(e):
            r = e * S2
            for dlo in range(D // LANES):
                sl = pl.ds(dlo * LANES, LANES)
                v = rows2_v[r, sl]
                for q in range(1, S2):
                    v = v + rows2_v[r + q, sl]
                acc_v[e, sl] = v

        pltpu.sync_copy(acc_v, f2s_hbm.at[pl.ds(base1 + off, CH)])


def _sc_gather(features, adj, batch, node_type):
    mesh = plsc.VectorSubcoreMesh(core_axis_name="c", subcore_axis_name="s")
    f32, i32 = jnp.float32, jnp.int32
    kern = pl.kernel(
        _sc_body,
        compiler_params=dataclasses.replace(
            pltpu.CompilerParams(), needs_layout_passes=False),
        out_type=[
            jax.ShapeDtypeStruct((B, D), f32),
            jax.ShapeDtypeStruct((B * S1, D), f32),
            jax.ShapeDtypeStruct((B * S1, D), f32),
            jax.ShapeDtypeStruct((B,), i32),
            jax.ShapeDtypeStruct((B * S1,), i32),
            jax.ShapeDtypeStruct((B * S1 * S2,), i32),
        ],
        mesh=mesh,
        scratch_types=[
            pltpu.VMEM((BPW,), i32),
            pltpu.VMEM((BPW, ADJP), i32),
            pltpu.VMEM((S1W,), i32),
            pltpu.VMEM((CH, ADJP), i32),
            pltpu.VMEM((CH * S2,), i32),
            pltpu.VMEM((CH * S2, D), f32),
            pltpu.VMEM((CH, D), f32),
            pltpu.VMEM((CH, D), f32),
            pltpu.VMEM((S1W,), i32),
            pltpu.VMEM((CH * S2,), i32),
            pltpu.VMEM((BPW, D), f32),
            pltpu.VMEM((BPW,), i32),
        ],
    )
    return kern(features, adj, batch, node_type)


R1BLK = 1600
G1BLK = R1BLK // S1


def _tc1_body(f1_r, f2s_r, nt1_r, nt2_r, te_r, ws0_r, wn0_r,
              h1m_r, f1m_r, te1m_r):
    f32 = jnp.float32
    te = te_r[...]
    ws0 = ws0_r[...]
    wn0 = wn0_r[...]
    wsf, wst = ws0[:D], ws0[D:D + TD] + ws0[D + TD:]
    wnf, wnt = wn0[:D], wn0[D:D + TD] + wn0[D + TD:]

    nt1 = nt1_r[...]
    te1 = jnp.zeros((R1BLK, TD), f32)
    for t in range(4):
        te1 = te1 + jnp.where(nt1 == t, 1.0, 0.0) * te[t][None, :]

    nt2 = nt2_r[...]
    te2s = jnp.zeros((R1BLK, TD), f32)
    for t in range(4):
        cnt = jnp.sum(jnp.where(nt2 == t, 1.0, 0.0), axis=1, keepdims=True)
        te2s = te2s + cnt * te[t][None, :]

    f1 = f1_r[...]
    p_s = (jnp.dot(f1, wsf, preferred_element_type=f32)
           + jnp.dot(te1, wst, preferred_element_type=f32))
    p_n = (jnp.dot(f2s_r[...] * (1.0 / S2), wnf, preferred_element_type=f32)
           + jnp.dot(te2s * (1.0 / S2), wnt, preferred_element_type=f32))
    h1 = jnp.maximum(jnp.concatenate([p_s, p_n], axis=1), 0.0)

    h1m_r[...] = jnp.mean(h1.reshape(G1BLK, S1, 2 * HID), axis=1)
    f1m_r[...] = jnp.mean(f1.reshape(G1BLK, S1, D), axis=1)
    te1m_r[...] = jnp.mean(te1.reshape(G1BLK, S1, TD), axis=1)


def _tc_stage1(f1, f2s, nt1, nt2, te, ws0, wn0):
    f32 = jnp.float32
    n_steps = (B * S1) // R1BLK
    full = lambda shape: pl.BlockSpec(shape, lambda i: (0, 0))
    return pl.pallas_call(
        _tc1_body,
        grid=(n_steps,),
        in_specs=[
            pl.BlockSpec((R1BLK, D), lambda i: (i, 0)),
            pl.BlockSpec((R1BLK, D), lambda i: (i, 0)),
            pl.BlockSpec((R1BLK, 1), lambda i: (i, 0)),
            pl.BlockSpec((R1BLK, S2), lambda i: (i, 0)),
            full((4, TD)),
            full((D + 2 * TD, HID)),
            full((D + 2 * TD, HID)),
        ],
        out_specs=[
            pl.BlockSpec((G1BLK, 2 * HID), lambda i: (i, 0)),
            pl.BlockSpec((G1BLK, D), lambda i: (i, 0)),
            pl.BlockSpec((G1BLK, TD), lambda i: (i, 0)),
        ],
        out_shape=[
            jax.ShapeDtypeStruct((B, 2 * HID), f32),
            jax.ShapeDtypeStruct((B, D), f32),
            jax.ShapeDtypeStruct((B, TD), f32),
        ],
    )(f1, f2s, nt1, nt2, te, ws0, wn0)


def _tc2_body(f0_r, nt0_r, h1m_r, f1m_r, te1m_r, te_r,
              ws0_r, wn0_r, ws1_r, wn1_r, out_r):
    f32 = jnp.float32
    te = te_r[...]
    ws0 = ws0_r[...]
    wn0 = wn0_r[...]
    wsf, wst = ws0[:D], ws0[D:D + TD] + ws0[D + TD:]
    wnf, wnt = wn0[:D], wn0[D:D + TD] + wn0[D + TD:]

    nt0 = nt0_r[...]
    te0 = jnp.zeros((B, TD), f32)
    for t in range(4):
        te0 = te0 + jnp.where(nt0 == t, 1.0, 0.0) * te[t][None, :]

    p_s = (jnp.dot(f0_r[...], wsf, preferred_element_type=f32)
           + jnp.dot(te0, wst, preferred_element_type=f32))
    p_n = (jnp.dot(f1m_r[...], wnf, preferred_element_type=f32)
           + jnp.dot(te1m_r[...], wnt, preferred_element_type=f32))
    h0 = jnp.maximum(jnp.concatenate([p_s, p_n], axis=1), 0.0)

    out = jnp.concatenate(
        [jnp.dot(h0, ws1_r[...], preferred_element_type=f32),
         jnp.dot(h1m_r[...], wn1_r[...], preferred_element_type=f32)], axis=1)
    norm = jnp.sqrt(jnp.sum(out * out, axis=1, keepdims=True))
    out_r[...] = out / (norm + 1e-12)


def _tc_stage2(f0, nt0, h1m, f1m, te1m, te, ws0, wn0, ws1, wn1):
    return pl.pallas_call(
        _tc2_body,
        out_shape=jax.ShapeDtypeStruct((B, D), jnp.float32),
    )(f0, nt0, h1m, f1m, te1m, te, ws0, wn0, ws1, wn1)


def kernel(features, node_type, adj, batch, type_embeds,
           W_self_0, W_neigh_0, W_self_1, W_neigh_1):
    nt = node_type.astype(jnp.int32)
    adj_i = adj.astype(jnp.int32)
    batch_i = batch.astype(jnp.int32)

    adj_p = jnp.pad(adj_i, ((0, 0), (0, ADJP - MAX_DEG)))
    f0, f1, f2s, nt0, nt1, nt2 = _sc_gather(features, adj_p, batch_i, nt)

    h1m, f1m, te1m = _tc_stage1(
        f1, f2s, nt1.reshape(B * S1, 1), nt2.reshape(B * S1, S2),
        type_embeds, W_self_0, W_neigh_0)

    return _tc_stage2(f0, nt0.reshape(B, 1), h1m, f1m, te1m,
                      type_embeds, W_self_0, W_neigh_0, W_self_1, W_neigh_1)


# speedup vs baseline: 3.3169x; 1.1044x over previous
import dataclasses
import functools

import jax
import jax.numpy as jnp
from jax import lax
from jax.experimental import pallas as pl
from jax.experimental.pallas import tpu as pltpu
from jax.experimental.pallas import tpu_sc as plsc

N = 100000
D = 128
B = 512
MAX_DEG = 25
S1, S2 = 25, 10
HID = 64
TD = 16

ADJP = 128
NW = 32
BPW = B // NW
S1W = BPW * S1
CH = 40
NCH = S1W // CH
LANES = 16


def _flatten_rows(src_ref, dst_ref, count, fanout):
    lane = lax.iota(jnp.int32, LANES)
    for k in range((count * fanout) // LANES):
        p = k * LANES + lane
        e = p // fanout
        q = p - e * fanout
        dst_ref[pl.ds(k * LANES, LANES)] = plsc.load_gather(src_ref, [e, q])


def _sc_body(feat_hbm, adj_hbm, batch_hbm, nt_hbm,
             f0_hbm, f1_hbm, f2s_hbm, nt0_hbm, nt1_hbm, nt2_hbm,
             idx0_v, adjr1_v, idx1_v, idx2_v,
             rows2a_v, rows2b_v, rows1_v, acc_v, nt1_v, nt2_v,
             sema, semb):
    w = lax.axis_index("s") * 2 + lax.axis_index("c")
    base0 = w * BPW
    base1 = w * S1W
    base2 = w * S1W * S2

    pltpu.sync_copy(batch_hbm.at[pl.ds(base0, BPW)], idx0_v)
    pltpu.sync_copy(feat_hbm.at[idx0_v], rows1_v.at[pl.ds(0, BPW)])
    pltpu.sync_copy(nt_hbm.at[idx0_v], nt1_v.at[pl.ds(0, BPW)])
    pltpu.sync_copy(rows1_v.at[pl.ds(0, BPW)], f0_hbm.at[pl.ds(base0, BPW)])
    pltpu.sync_copy(nt1_v.at[pl.ds(0, BPW)], nt0_hbm.at[pl.ds(base0, BPW)])

    pltpu.sync_copy(adj_hbm.at[idx0_v, :], adjr1_v.at[pl.ds(0, BPW)])
    _flatten_rows(adjr1_v, idx1_v, BPW, S1)
    pltpu.sync_copy(nt_hbm.at[idx1_v], nt1_v)
    pltpu.sync_copy(nt1_v, nt1_hbm.at[pl.ds(base1, S1W)])

    for c in range(NCH):
        off = c * CH
        pltpu.sync_copy(adj_hbm.at[idx1_v.at[pl.ds(off, CH)], :], adjr1_v)
        _flatten_rows(adjr1_v, idx2_v.at[pl.ds(off * S2, CH * S2)], CH, S2)
    pltpu.sync_copy(nt_hbm.at[idx2_v], nt2_v)
    pltpu.sync_copy(nt2_v, nt2_hbm.at[pl.ds(base2, S1W * S2)])

    bufs = (rows2a_v, rows2b_v)
    sems = (sema, semb)

    def start_fetch(c):
        return pltpu.async_copy(
            feat_hbm.at[idx2_v.at[pl.ds(c * CH * S2, CH * S2)]],
            bufs[c % 2], sems[c % 2])

    pending = start_fetch(0)
    for c in range(NCH):
        nxt = start_fetch(c + 1) if c + 1 < NCH else None
        off = c * CH
        pltpu.sync_copy(feat_hbm.at[idx1_v.at[pl.ds(off, CH)]], rows1_v)
        pltpu.sync_copy(rows1_v, f1_hbm.at[pl.ds(base1 + off, CH)])

        pending.wait()
        rows2_v = bufs[c % 2]

        @pl.loop(0, CH)
        def _acc(e):
            r = e * S2
            for dlo in range(D // LANES):
                sl = pl.ds(dlo * LANES, LANES)
                v = rows2_v[r, sl]
                for q in range(1, S2):
                    v = v + rows2_v[r + q, sl]
                acc_v[e, sl] = v

        pltpu.sync_copy(acc_v, f2s_hbm.at[pl.ds(base1 + off, CH)])
        pending = nxt


def _sc_gather(features, adj, batch, node_type):
    mesh = plsc.VectorSubcoreMesh(core_axis_name="c", subcore_axis_name="s")
    f32, i32 = jnp.float32, jnp.int32
    kern = pl.kernel(
        _sc_body,
        compiler_params=dataclasses.replace(
            pltpu.CompilerParams(), needs_layout_passes=False),
        out_type=[
            jax.ShapeDtypeStruct((B, D), f32),
            jax.ShapeDtypeStruct((B * S1, D), f32),
            jax.ShapeDtypeStruct((B * S1, D), f32),
            jax.ShapeDtypeStruct((B,), i32),
            jax.ShapeDtypeStruct((B * S1,), i32),
            jax.ShapeDtypeStruct((B * S1 * S2,), i32),
        ],
        mesh=mesh,
        scratch_types=[
            pltpu.VMEM((BPW,), i32),
            pltpu.VMEM((CH, ADJP), i32),
            pltpu.VMEM((S1W,), i32),
            pltpu.VMEM((S1W * S2,), i32),
            pltpu.VMEM((CH * S2, D), f32),
            pltpu.VMEM((CH * S2, D), f32),
            pltpu.VMEM((CH, D), f32),
            pltpu.VMEM((CH, D), f32),
            pltpu.VMEM((S1W,), i32),
            pltpu.VMEM((S1W * S2,), i32),
            pltpu.SemaphoreType.DMA,
            pltpu.SemaphoreType.DMA,
        ],
    )
    return kern(features, adj, batch, node_type)


R1BLK = 1600
G1BLK = R1BLK // S1


def _tc1_body(f1_r, f2s_r, nt1_r, nt2_r, te_r, ws0_r, wn0_r,
              h1m_r, f1m_r, te1m_r):
    f32 = jnp.float32
    te = te_r[...]
    ws0 = ws0_r[...]
    wn0 = wn0_r[...]
    wsf, wst = ws0[:D], ws0[D:D + TD] + ws0[D + TD:]
    wnf, wnt = wn0[:D], wn0[D:D + TD] + wn0[D + TD:]

    nt1 = nt1_r[...]
    te1 = jnp.zeros((R1BLK, TD), f32)
    for t in range(4):
        te1 = te1 + jnp.where(nt1 == t, 1.0, 0.0) * te[t][None, :]

    nt2 = nt2_r[...]
    te2s = jnp.zeros((R1BLK, TD), f32)
    for t in range(4):
        cnt = jnp.sum(jnp.where(nt2 == t, 1.0, 0.0), axis=1, keepdims=True)
        te2s = te2s + cnt * te[t][None, :]

    bf16 = jnp.bfloat16
    f1 = f1_r[...]
    p_s = (jnp.dot(f1.astype(bf16), wsf.astype(bf16),
                   preferred_element_type=f32)
           + jnp.dot(te1, wst, preferred_element_type=f32))
    p_n = (jnp.dot((f2s_r[...] * (1.0 / S2)).astype(bf16), wnf.astype(bf16),
                   preferred_element_type=f32)
           + jnp.dot(te2s * (1.0 / S2), wnt, preferred_element_type=f32))
    h1 = jnp.maximum(jnp.concatenate([p_s, p_n], axis=1), 0.0)

    h1m_r[...] = jnp.mean(h1.reshape(G1BLK, S1, 2 * HID), axis=1)
    f1m_r[...] = jnp.mean(f1.reshape(G1BLK, S1, D), axis=1)
    te1m_r[...] = jnp.mean(te1.reshape(G1BLK, S1, TD), axis=1)


def _tc_stage1(f1, f2s, nt1, nt2, te, ws0, wn0):
    f32 = jnp.float32
    n_steps = (B * S1) // R1BLK
    full = lambda shape: pl.BlockSpec(shape, lambda i: (0, 0))
    return pl.pallas_call(
        _tc1_body,
        grid=(n_steps,),
        in_specs=[
            pl.BlockSpec((R1BLK, D), lambda i: (i, 0)),
            pl.BlockSpec((R1BLK, D), lambda i: (i, 0)),
            pl.BlockSpec((R1BLK, 1), lambda i: (i, 0)),
            pl.BlockSpec((R1BLK, S2), lambda i: (i, 0)),
            full((4, TD)),
            full((D + 2 * TD, HID)),
            full((D + 2 * TD, HID)),
        ],
        out_specs=[
            pl.BlockSpec((G1BLK, 2 * HID), lambda i: (i, 0)),
            pl.BlockSpec((G1BLK, D), lambda i: (i, 0)),
            pl.BlockSpec((G1BLK, TD), lambda i: (i, 0)),
        ],
        out_shape=[
            jax.ShapeDtypeStruct((B, 2 * HID), f32),
            jax.ShapeDtypeStruct((B, D), f32),
            jax.ShapeDtypeStruct((B, TD), f32),
        ],
    )(f1, f2s, nt1, nt2, te, ws0, wn0)


def _tc2_body(f0_r, nt0_r, h1m_r, f1m_r, te1m_r, te_r,
              ws0_r, wn0_r, ws1_r, wn1_r, out_r):
    f32 = jnp.float32
    te = te_r[...]
    ws0 = ws0_r[...]
    wn0 = wn0_r[...]
    wsf, wst = ws0[:D], ws0[D:D + TD] + ws0[D + TD:]
    wnf, wnt = wn0[:D], wn0[D:D + TD] + wn0[D + TD:]

    nt0 = nt0_r[...]
    te0 = jnp.zeros((B, TD), f32)
    for t in range(4):
        te0 = te0 + jnp.where(nt0 == t, 1.0, 0.0) * te[t][None, :]

    p_s = (jnp.dot(f0_r[...], wsf, preferred_element_type=f32)
           + jnp.dot(te0, wst, preferred_element_type=f32))
    p_n = (jnp.dot(f1m_r[...], wnf, preferred_element_type=f32)
           + jnp.dot(te1m_r[...], wnt, preferred_element_type=f32))
    h0 = jnp.maximum(jnp.concatenate([p_s, p_n], axis=1), 0.0)

    out = jnp.concatenate(
        [jnp.dot(h0, ws1_r[...], preferred_element_type=f32),
         jnp.dot(h1m_r[...], wn1_r[...], preferred_element_type=f32)], axis=1)
    norm = jnp.sqrt(jnp.sum(out * out, axis=1, keepdims=True))
    out_r[...] = out / (norm + 1e-12)


def _tc_stage2(f0, nt0, h1m, f1m, te1m, te, ws0, wn0, ws1, wn1):
    return pl.pallas_call(
        _tc2_body,
        out_shape=jax.ShapeDtypeStruct((B, D), jnp.float32),
    )(f0, nt0, h1m, f1m, te1m, te, ws0, wn0, ws1, wn1)


def kernel(features, node_type, adj, batch, type_embeds,
           W_self_0, W_neigh_0, W_self_1, W_neigh_1):
    nt = node_type.astype(jnp.int32)
    adj_i = adj.astype(jnp.int32)
    batch_i = batch.astype(jnp.int32)

    adj_p = jnp.pad(adj_i, ((0, 0), (0, ADJP - MAX_DEG)))
    f0, f1, f2s, nt0, nt1, nt2 = _sc_gather(features, adj_p, batch_i, nt)

    h1m, f1m, te1m = _tc_stage1(
        f1, f2s, nt1.reshape(B * S1, 1), nt2.reshape(B * S1, S2),
        type_embeds, W_self_0, W_neigh_0)

    return _tc_stage2(f0, nt0.reshape(B, 1), h1m, f1m, te1m,
                      type_embeds, W_self_0, W_neigh_0, W_self_1, W_neigh_1)
